# baseline (speedup 1.0000x reference)
import functools

import jax
import jax.numpy as jnp
from jax import lax
from jax.experimental import pallas as pl
from jax.experimental.pallas import tpu as pltpu
from jax.experimental.pallas import tpu_sc as plsc

N_EDGES = 320000
N_NODES = 10000
D = 128

NC = 2
NS = 16
NW = NC * NS

CH = 128
SC_E = 128
NSCH = N_EDGES // SC_E
NSCH_PAD = 2528

HALF = N_NODES // NC
TR = HALF
ACC_ROWS = HALF + 8

PIDX = 32
P = 24

ZROWS = 312
ZLAST = HALF - 15 * ZROWS

_sc_mesh = plsc.VectorSubcoreMesh(core_axis_name="c", subcore_axis_name="s")


@functools.partial(
    pl.kernel,
    out_type=jax.ShapeDtypeStruct((N_NODES, D), jnp.float32),
    mesh=_sc_mesh,
    scratch_types=[
        pltpu.VMEM((PIDX, SC_E), jnp.int32),
        pltpu.VMEM((NW, CH), jnp.int32),
        pltpu.VMEM((SC_E, D), jnp.float32),
        pltpu.VMEM((SC_E, D), jnp.float32),
        pltpu.VMEM_SHARED((ACC_ROWS, D), jnp.float32),
        pltpu.SemaphoreType.DMA,
        pltpu.SemaphoreType.DMA,
    ],
)
def _sc_segment_sum(x_hbm, idx_hbm, bnd_hbm, out_hbm, idxbuf, bndbuf,
                    xbuf0, xbuf1, acc, fsem0, fsem1):
    c = lax.axis_index("c")
    s = lax.axis_index("s")
    w = c * NS + s
    base = c * HALF

    pltpu.sync_copy(bnd_hbm, bndbuf)
    bv = bndbuf[w, pl.ds(0, 16)]
    lo = bv[0]
    cnt = bv[1]
    npages = bv[2]

    def zero_row(r, _):
        for g in range(D // 16):
            xbuf0[r, pl.ds(g * 16, 16)] = jnp.zeros((16,), jnp.float32)
        return 0
    lax.fori_loop(0, SC_E, zero_row, 0)
    zr0 = s * ZROWS
    for k in range(ZROWS // SC_E):
        pltpu.sync_copy(xbuf0, acc.at[pl.ds(zr0 + k * SC_E, SC_E)])
    zrem = ZROWS % SC_E
    pltpu.sync_copy(xbuf0.at[pl.ds(0, zrem)],
                    acc.at[pl.ds(zr0 + ZROWS - zrem, zrem)])

    @pl.when(s == NS - 1)
    def _zero_last():
        pltpu.sync_copy(xbuf0.at[pl.ds(0, ZLAST - ZROWS)],
                        acc.at[pl.ds(zr0 + ZROWS, ZLAST - ZROWS)])

    plsc.subcore_barrier()

    def fetch(sc, buf, sem):
        row = pl.multiple_of(jnp.minimum(sc, NSCH - 1) * SC_E, 8)
        pltpu.async_copy(x_hbm.at[pl.ds(row, SC_E)], buf, sem)

    def fetch_wait(buf, sem):
        pltpu.make_async_copy(x_hbm.at[pl.ds(0, SC_E)], buf, sem).wait()

    def page(p, _):
        pstart = lo + p * P
        pal = pl.multiple_of((pstart // 8) * 8, 8)
        poff = pstart - pal
        pltpu.sync_copy(idx_hbm.at[pl.ds(pal, PIDX)], idxbuf)

        def loc_row(r, _):
            for g in range(SC_E // 16):
                v = idxbuf[r, pl.ds(g * 16, 16)] - base
                bad = (v < 0) | (v >= HALF)
                idxbuf[r, pl.ds(g * 16, 16)] = jnp.where(bad, TR, v)
            return 0
        lax.fori_loop(0, PIDX, loc_row, 0)

        m = jnp.minimum(P, cnt - p * P)
        fetch(pstart, xbuf0, fsem0)
        fetch(pstart + 1, xbuf1, fsem1)

        def chunk_body(k, _):
            @pl.when(k % 2 == 0)
            def _even():
                fetch_wait(xbuf0, fsem0)
                pltpu.sync_copy(xbuf0,
                                acc.at[idxbuf.at[poff + k]],
                                add=True)
                fetch(pstart + k + 2, xbuf0, fsem0)

            @pl.when(k % 2 == 1)
            def _odd():
                fetch_wait(xbuf1, fsem1)
                pltpu.sync_copy(xbuf1,
                                acc.at[idxbuf.at[poff + k]],
                                add=True)
                fetch(pstart + k + 2, xbuf1, fsem1)
            return 0
        lax.fori_loop(0, m, chunk_body, 0)

        fetch_wait(xbuf0, fsem0)
        fetch_wait(xbuf1, fsem1)
        return 0

    lax.fori_loop(0, npages, page, 0)
    plsc.subcore_barrier()

    out0 = base + zr0
    pltpu.sync_copy(acc.at[pl.ds(zr0, ZROWS)], out_hbm.at[pl.ds(out0, ZROWS)])

    @pl.when(s == NS - 1)
    def _copy_last():
        pltpu.sync_copy(acc.at[pl.ds(zr0 + ZROWS, ZLAST - ZROWS)],
                        out_hbm.at[pl.ds(out0 + ZROWS, ZLAST - ZROWS)])


def _mlp_body(a_ref, w1_ref, b1_ref, w2_ref, b2_ref, o_ref):
    h = lax.dot_general(a_ref[...], w1_ref[...], (((1,), (1,)), ((), ())),
                        preferred_element_type=jnp.float32)
    h = jnp.maximum(h + b1_ref[...], 0.0)
    o = lax.dot_general(h, w2_ref[...], (((1,), (1,)), ((), ())),
                        preferred_element_type=jnp.float32)
    o_ref[...] = o + b2_ref[...]


ROW_BLK = 1000
N_BLKS = N_NODES // ROW_BLK


def _mlp(agg, W1, b1, W2, b2):
    return pl.pallas_call(
        _mlp_body,
        grid=(N_BLKS,),
        in_specs=[
            pl.BlockSpec((ROW_BLK, D), lambda i: (i, 0)),
            pl.BlockSpec((D, D), lambda i: (0, 0)),
            pl.BlockSpec((1, D), lambda i: (0, 0)),
            pl.BlockSpec((D, D), lambda i: (0, 0)),
            pl.BlockSpec((1, D), lambda i: (0, 0)),
        ],
        out_specs=pl.BlockSpec((ROW_BLK, D), lambda i: (i, 0)),
        out_shape=jax.ShapeDtypeStruct((N_NODES, D), jnp.float32),
    )(agg, W1, b1.reshape(1, D), W2, b2.reshape(1, D))


def kernel(x, index, W1, b1, W2, b2):
    idx = index.astype(jnp.int32)
    idxm = jnp.pad(idx.reshape(NSCH, SC_E), ((0, NSCH_PAD - NSCH), (0, 0)))

    e1 = jnp.searchsorted(idx, jnp.int32(HALF)).astype(jnp.int32)
    e = jnp.stack([jnp.int32(0), e1, jnp.int32(N_EDGES)])
    sc_lo = e[:-1] // SC_E
    sc_hi = (e[1:] + SC_E - 1) // SC_E
    n2 = sc_hi - sc_lo
    sids = jnp.arange(NS, dtype=jnp.int32)
    q = n2[:, None] // NS
    rem = n2[:, None] % NS
    tile_lo = sc_lo[:, None] + sids[None, :] * q + jnp.minimum(sids[None, :],
                                                               rem)
    tile_cnt = q + (sids[None, :] < rem).astype(jnp.int32)
    tile_lo = tile_lo.reshape(NW, 1)
    tile_cnt = tile_cnt.reshape(NW, 1)
    tile_np = (tile_cnt + P - 1) // P
    bnd = jnp.concatenate([
        tile_lo, tile_cnt, tile_np,
        jnp.zeros((NW, CH - 3), jnp.int32),
    ], axis=1)

    agg = _sc_segment_sum(x, idxm, bnd)
    return _mlp(agg, W1, b1, W2, b2)


# speedup vs baseline: 1.3976x; 1.3976x over previous
import functools

import jax
import jax.numpy as jnp
from jax import lax
from jax.experimental import pallas as pl
from jax.experimental.pallas import tpu as pltpu
from jax.experimental.pallas import tpu_sc as plsc

N_EDGES = 320000
N_NODES = 10000
D = 128

NC = 2
NS = 16

CH = 128
NCH = N_EDGES // CH
PCC = NCH // NC
BASE = PCC // NS
REM = PCC - BASE * NS
MAXCH = BASE + 1
MAXCHA = 88
NCH_PAD = 2504

ZROWS = 78
ROWS_PER_TILE = 8 * ZROWS
TAIL0 = NS * ROWS_PER_TILE
TAIL = N_NODES - TAIL0

_sc_mesh = plsc.VectorSubcoreMesh(core_axis_name="c", subcore_axis_name="s")


@functools.partial(
    pl.kernel,
    out_type=jax.ShapeDtypeStruct((NC * N_NODES, D), jnp.float32),
    mesh=_sc_mesh,
    scratch_types=[
        pltpu.VMEM((CH, D), jnp.float32),
        pltpu.VMEM((CH, D), jnp.float32),
        pltpu.VMEM((MAXCHA, CH), jnp.int32),
        pltpu.VMEM_SHARED((N_NODES, D), jnp.float32),
        pltpu.SemaphoreType.DMA,
        pltpu.SemaphoreType.DMA,
    ],
)
def _sc_segment_sum(x_hbm, idx_hbm, out_hbm, xbuf0, xbuf1, idxbuf,
                    acc, fsem0, fsem1):
    c = lax.axis_index("c")
    s = lax.axis_index("s")
    start = c * PCC + s * BASE + jnp.minimum(s, REM)
    n_chunks = BASE + (s < REM).astype(jnp.int32)

    start_al = pl.multiple_of((start // 8) * 8, 8)
    off = start - start_al
    pltpu.async_copy(x_hbm.at[pl.ds(pl.multiple_of(start * CH, 8), CH)],
                     xbuf0, fsem0)
    pltpu.async_copy(idx_hbm.at[pl.ds(start_al, MAXCHA)], idxbuf, fsem1)

    def zero_row(r, _):
        for j in range(D // 16):
            xbuf1[r, pl.ds(j * 16, 16)] = jnp.zeros((16,), jnp.float32)
        return 0
    lax.fori_loop(0, CH, zero_row, 0)
    row0 = s * ROWS_PER_TILE
    for k in range(ROWS_PER_TILE // CH):
        pltpu.sync_copy(xbuf1, acc.at[pl.ds(row0 + k * CH, CH)])
    zrem = ROWS_PER_TILE - (ROWS_PER_TILE // CH) * CH
    pltpu.sync_copy(xbuf1.at[pl.ds(0, zrem)],
                    acc.at[pl.ds(row0 + ROWS_PER_TILE - zrem, zrem)])

    @pl.when(s == NS - 1)
    def _zero_tail():
        pltpu.sync_copy(xbuf1.at[pl.ds(0, TAIL)], acc.at[pl.ds(TAIL0, TAIL)])

    pltpu.make_async_copy(idx_hbm.at[pl.ds(0, MAXCHA)], idxbuf, fsem1).wait()
    plsc.subcore_barrier()

    def fetch(chunk, buf, sem):
        row = pl.multiple_of(jnp.minimum(chunk, NCH - 1) * CH, 8)
        pltpu.async_copy(x_hbm.at[pl.ds(row, CH)], buf, sem)

    def fetch_wait(buf, sem):
        pltpu.make_async_copy(x_hbm.at[pl.ds(0, CH)], buf, sem).wait()

    fetch(start + 1, xbuf1, fsem1)

    def body(i, _):
        j0 = 2 * i
        fetch_wait(xbuf0, fsem0)
        pltpu.sync_copy(xbuf0, acc.at[idxbuf.at[off + j0]], add=True)
        fetch(start + j0 + 2, xbuf0, fsem0)
        fetch_wait(xbuf1, fsem1)
        pltpu.sync_copy(xbuf1, acc.at[idxbuf.at[off + j0 + 1]], add=True)
        fetch(start + j0 + 3, xbuf1, fsem1)
        return 0
    lax.fori_loop(0, BASE // 2, body, 0)

    fetch_wait(xbuf0, fsem0)
    fetch_wait(xbuf1, fsem1)

    @pl.when(s < REM)
    def _tail_chunk():
        pltpu.sync_copy(xbuf0, acc.at[idxbuf.at[off + BASE]], add=True)

    plsc.subcore_barrier()

    out0 = c * N_NODES + row0
    pltpu.sync_copy(acc.at[pl.ds(row0, ROWS_PER_TILE)],
                    out_hbm.at[pl.ds(out0, ROWS_PER_TILE)])

    @pl.when(s == NS - 1)
    def _copy_tail():
        pltpu.sync_copy(acc.at[pl.ds(TAIL0, TAIL)],
                        out_hbm.at[pl.ds(c * N_NODES + TAIL0, TAIL)])


def _mlp_body(a_ref, b_ref, w1_ref, b1_ref, w2_ref, b2_ref, o_ref):
    ssum = a_ref[...] + b_ref[...]
    h = lax.dot_general(ssum, w1_ref[...], (((1,), (1,)), ((), ())),
                        preferred_element_type=jnp.float32)
    h = jnp.maximum(h + b1_ref[...], 0.0)
    o = lax.dot_general(h, w2_ref[...], (((1,), (1,)), ((), ())),
                        preferred_element_type=jnp.float32)
    o_ref[...] = o + b2_ref[...]


ROW_BLK = 2000
N_BLKS = N_NODES // ROW_BLK


def _mlp(partials, W1, b1, W2, b2):
    return pl.pallas_call(
        _mlp_body,
        grid=(N_BLKS,),
        in_specs=[
            pl.BlockSpec((ROW_BLK, D), lambda i: (i, 0)),
            pl.BlockSpec((ROW_BLK, D), lambda i: (i + N_BLKS, 0)),
            pl.BlockSpec((D, D), lambda i: (0, 0)),
            pl.BlockSpec((1, D), lambda i: (0, 0)),
            pl.BlockSpec((D, D), lambda i: (0, 0)),
            pl.BlockSpec((1, D), lambda i: (0, 0)),
        ],
        out_specs=pl.BlockSpec((ROW_BLK, D), lambda i: (i, 0)),
        out_shape=jax.ShapeDtypeStruct((N_NODES, D), jnp.float32),
    )(partials, partials, W1, b1.reshape(1, D), W2, b2.reshape(1, D))


def kernel(x, index, W1, b1, W2, b2):
    idx = index.astype(jnp.int32).reshape(NCH, CH)
    idx = jnp.pad(idx, ((0, NCH_PAD - NCH), (0, 0)))
    partials = _sc_segment_sum(x, idx)
    return _mlp(partials, W1, b1, W2, b2)
